# baseline (speedup 1.0000x reference)
import functools

import jax
import jax.numpy as jnp
from jax import lax
from jax.experimental import pallas as pl
from jax.experimental.pallas import tpu as pltpu
from jax.experimental.pallas import tpu_sc as plsc

EMBED_DIM = 1024
NBINS = 6
BATCH = 16384
NC = 2
NS = 16
NW = NC * NS
BPW = BATCH // NW
CHUNK = 32
NCHUNK = BPW // CHUNK


def _sc_body(xr, th, ts, tt, oh, os_, ot, idx_v, buf, gsem):
    wid = lax.axis_index("s") * NC + lax.axis_index("c")
    base = wid * BPW
    for t, (tab, out) in enumerate(((th, oh), (ts, os_), (tt, ot))):
        pltpu.sync_copy(xr.at[t, wid], idx_v)

        def chunk_body(k, carry, tab=tab, out=out):
            pltpu.async_copy(tab.at[idx_v.at[k]], buf, gsem).wait()
            pltpu.sync_copy(buf, out.at[pl.ds(base + k * CHUNK, CHUNK)])
            return carry

        lax.fori_loop(0, NCHUNK, chunk_body, 0)


@jax.jit
def _sc_lookup(xr, helix_table, sheet_table, turns_table):
    mesh = plsc.VectorSubcoreMesh(core_axis_name="c", subcore_axis_name="s")
    out_t = [jax.ShapeDtypeStruct((BATCH, EMBED_DIM), jnp.float32)] * 3
    return pl.kernel(
        _sc_body,
        mesh=mesh,
        out_type=out_t,
        scratch_types=[
            pltpu.VMEM((NCHUNK, CHUNK), jnp.int32),
            pltpu.VMEM((CHUNK, EMBED_DIM), jnp.float32),
            pltpu.SemaphoreType.DMA,
        ],
    )(xr, helix_table, sheet_table, turns_table)


def kernel(x, helix_table, sheet_table, turns_table):
    xr = x.astype(jnp.int32).T.reshape(3, NW, NCHUNK, CHUNK)
    return tuple(_sc_lookup(xr, helix_table, sheet_table, turns_table))


# speedup vs baseline: 1.0051x; 1.0051x over previous
import functools

import jax
import jax.numpy as jnp
from jax import lax
from jax.experimental import pallas as pl
from jax.experimental.pallas import tpu as pltpu
from jax.experimental.pallas import tpu_sc as plsc

EMBED_DIM = 1024
NBINS = 6
BATCH = 16384
NC = 2
NS = 16
NW = NC * NS
BPW = BATCH // NW
CHUNK = 32
NCHUNK = BPW // CHUNK


def _sc_body(xr, th, ts, tt, oh, os_, ot, idx_v, buf0, buf1, gsem,
             osem0, osem1):
    wid = lax.axis_index("s") * NC + lax.axis_index("c")
    base = wid * BPW
    bufs = ((buf0, osem0), (buf1, osem1))
    for t, (tab, out) in enumerate(((th, oh), (ts, os_), (tt, ot))):
        pltpu.sync_copy(xr.at[t, wid], idx_v)

        def chunk_body(kk, carry, t=t, tab=tab, out=out):
            for b, (buf, osem) in enumerate(bufs):
                k = kk * 2 + b

                @pl.when(jnp.logical_or(kk > 0, t > 0))
                def _(buf=buf, osem=osem):
                    pltpu.make_async_copy(
                        buf, out.at[pl.ds(0, CHUNK)], osem).wait()

                pltpu.async_copy(tab.at[idx_v.at[k]], buf, gsem).wait()
                pltpu.async_copy(
                    buf, out.at[pl.ds(base + k * CHUNK, CHUNK)], osem)
            return carry

        lax.fori_loop(0, NCHUNK // 2, chunk_body, 0)
    for buf, osem in bufs:
        pltpu.make_async_copy(buf, ot.at[pl.ds(0, CHUNK)], osem).wait()


@jax.jit
def _sc_lookup(xr, helix_table, sheet_table, turns_table):
    mesh = plsc.VectorSubcoreMesh(core_axis_name="c", subcore_axis_name="s")
    out_t = [jax.ShapeDtypeStruct((BATCH, EMBED_DIM), jnp.float32)] * 3
    return pl.kernel(
        _sc_body,
        mesh=mesh,
        out_type=out_t,
        scratch_types=[
            pltpu.VMEM((NCHUNK, CHUNK), jnp.int32),
            pltpu.VMEM((CHUNK, EMBED_DIM), jnp.float32),
            pltpu.VMEM((CHUNK, EMBED_DIM), jnp.float32),
            pltpu.SemaphoreType.DMA,
            pltpu.SemaphoreType.DMA,
            pltpu.SemaphoreType.DMA,
        ],
    )(xr, helix_table, sheet_table, turns_table)


def kernel(x, helix_table, sheet_table, turns_table):
    xr = x.astype(jnp.int32).T.reshape(3, NW, NCHUNK, CHUNK)
    return tuple(_sc_lookup(xr, helix_table, sheet_table, turns_table))


# speedup vs baseline: 1.0577x; 1.0523x over previous
import functools

import jax
import jax.numpy as jnp
from jax import lax
from jax.experimental import pallas as pl
from jax.experimental.pallas import tpu as pltpu
from jax.experimental.pallas import tpu_sc as plsc

EMBED_DIM = 1024
NBINS = 6
BATCH = 16384
NC = 2
NS = 16
NW = NC * NS
BPW = BATCH // NW
CHUNK = 32
NCHUNK = BPW // CHUNK
NCH_ALL = 3 * NCHUNK


def _sc_body(xr, th, ts, tt, oh, os_, ot, idx_v, tab_v, buf, gsem,
             osem0, osem1):
    wid = lax.axis_index("s") * NC + lax.axis_index("c")
    base = wid * BPW
    outs = (oh, os_, ot)

    for t, tab in enumerate((th, ts, tt)):
        pltpu.sync_copy(tab, tab_v.at[pl.ds(t * NBINS * EMBED_DIM,
                                            NBINS * EMBED_DIM)])
    pltpu.sync_copy(xr.at[wid], idx_v)

    def chunk_pair(kk, carry):
        for b, osem in ((0, osem0), (1, osem1)):
            k = kk * 2 + b
            t = k // NCHUNK
            kt = k - t * NCHUNK

            @pl.when(kk > 0)
            def _(b=b, osem=osem):
                pltpu.make_async_copy(
                    buf.at[pl.ds(b * CHUNK * EMBED_DIM, CHUNK * EMBED_DIM)],
                    oh.at[pl.ds(0, CHUNK * EMBED_DIM)], osem).wait()

            def group(g, cc, b=b, k=k, t=t):
                v = idx_v[k * 2 + g]
                for j in range(16):
                    src0 = (t * NBINS + v[j]) * EMBED_DIM
                    dst0 = (b * CHUNK + g * 16 + j) * EMBED_DIM
                    row = tab_v.at[pl.ds(src0, EMBED_DIM)]
                    dst = buf.at[pl.ds(dst0, EMBED_DIM)]

                    def col(c, c2, row=row, dst=dst):
                        dst[pl.ds(c * 16, 16)] = row[pl.ds(c * 16, 16)]
                        return c2

                    lax.fori_loop(0, EMBED_DIM // 16, col, 0, unroll=8)
                return cc

            lax.fori_loop(0, CHUNK // 16, group, 0)

            src = buf.at[pl.ds(b * CHUNK * EMBED_DIM, CHUNK * EMBED_DIM)]
            for tt_ in range(3):
                @pl.when(t == tt_)
                def _(tt_=tt_, src=src, kt=kt, osem=osem):
                    pltpu.async_copy(
                        src,
                        outs[tt_].at[pl.ds((base + kt * CHUNK) * EMBED_DIM,
                                           CHUNK * EMBED_DIM)],
                        osem)
        return carry

    lax.fori_loop(0, NCH_ALL // 2, chunk_pair, 0)

    for b, osem in ((0, osem0), (1, osem1)):
        pltpu.make_async_copy(
            buf.at[pl.ds(b * CHUNK * EMBED_DIM, CHUNK * EMBED_DIM)],
            oh.at[pl.ds(0, CHUNK * EMBED_DIM)], osem).wait()


@jax.jit
def _sc_lookup(xr, helix_table, sheet_table, turns_table):
    mesh = plsc.VectorSubcoreMesh(core_axis_name="c", subcore_axis_name="s")
    out_t = [jax.ShapeDtypeStruct((BATCH * EMBED_DIM,), jnp.float32)] * 3
    return pl.kernel(
        _sc_body,
        mesh=mesh,
        out_type=out_t,
        scratch_types=[
            pltpu.VMEM((3 * BPW // 16, 16), jnp.int32),
            pltpu.VMEM((3 * NBINS * EMBED_DIM,), jnp.float32),
            pltpu.VMEM((2 * CHUNK * EMBED_DIM,), jnp.float32),
            pltpu.SemaphoreType.DMA,
            pltpu.SemaphoreType.DMA,
            pltpu.SemaphoreType.DMA,
        ],
    )(xr, helix_table.reshape(-1), sheet_table.reshape(-1),
      turns_table.reshape(-1))


def kernel(x, helix_table, sheet_table, turns_table):
    xr = (x.astype(jnp.int32).T.reshape(3, NW, BPW)
          .transpose(1, 0, 2).reshape(NW, 3 * BPW // 16, 16))
    oh, os_, ot = _sc_lookup(xr, helix_table, sheet_table, turns_table)
    shape = (BATCH, EMBED_DIM)
    return (oh.reshape(shape), os_.reshape(shape), ot.reshape(shape))


# speedup vs baseline: 10.5162x; 9.9425x over previous
import functools

import jax
import jax.numpy as jnp
from jax import lax
from jax.experimental import pallas as pl
from jax.experimental.pallas import tpu as pltpu
from jax.experimental.pallas import tpu_sc as plsc

EMBED_DIM = 1024
NBINS = 6
BATCH = 16384
NC = 2
NS = 16
NW = NC * NS
BPW = BATCH // NW
CHUNK = 32
NCHUNK = BPW // CHUNK
NCH_ALL = 3 * NCHUNK


def _sc_body(xr, th, ts, tt, oh, os_, ot, idx_v, tab_v, buf, gsem,
             osem0, osem1):
    wid = lax.axis_index("s") * NC + lax.axis_index("c")
    base = wid * BPW
    outs = (oh, os_, ot)

    for t, tab in enumerate((th, ts, tt)):
        pltpu.sync_copy(tab, tab_v.at[pl.ds(t * NBINS * EMBED_DIM,
                                            NBINS * EMBED_DIM)])
    pltpu.sync_copy(xr.at[wid], idx_v)

    def chunk_pair(kk, carry):
        for b, osem in ((0, osem0), (1, osem1)):
            k = kk * 2 + b
            t = k // NCHUNK
            kt = k - t * NCHUNK

            @pl.when(kk > 0)
            def _(b=b, osem=osem):
                pltpu.make_async_copy(
                    buf.at[pl.ds(b * CHUNK * EMBED_DIM, CHUNK * EMBED_DIM)],
                    oh.at[pl.ds(0, CHUNK * EMBED_DIM)], osem).wait()

            def group(g, cc, b=b, k=k, t=t):
                v = idx_v[k * 2 + g]
                for j in range(16):
                    src0 = (t * NBINS + v[j]) * EMBED_DIM
                    dst0 = (b * CHUNK + g * 16 + j) * EMBED_DIM
                    row = tab_v.at[pl.ds(src0, EMBED_DIM)]
                    dst = buf.at[pl.ds(dst0, EMBED_DIM)]

                    def col(c, c2, row=row, dst=dst):
                        dst[pl.ds(c * 16, 16)] = row[pl.ds(c * 16, 16)]
                        return c2

                    lax.fori_loop(0, EMBED_DIM // 16, col, 0, unroll=8)
                return cc

            lax.fori_loop(0, CHUNK // 16, group, 0)

            src = buf.at[pl.ds(b * CHUNK * EMBED_DIM, CHUNK * EMBED_DIM)]
            for tt_ in range(3):
                @pl.when(t == tt_)
                def _(tt_=tt_, src=src, kt=kt, osem=osem):
                    pltpu.async_copy(
                        src,
                        outs[tt_].at[pl.ds((base + kt * CHUNK) * EMBED_DIM,
                                           CHUNK * EMBED_DIM)],
                        osem)
        return carry

    lax.fori_loop(0, NCH_ALL // 2, chunk_pair, 0)

    for b, osem in ((0, osem0), (1, osem1)):
        pltpu.make_async_copy(
            buf.at[pl.ds(b * CHUNK * EMBED_DIM, CHUNK * EMBED_DIM)],
            oh.at[pl.ds(0, CHUNK * EMBED_DIM)], osem).wait()


@jax.jit
def _sc_lookup(xr, helix_table, sheet_table, turns_table):
    mesh = plsc.VectorSubcoreMesh(core_axis_name="c", subcore_axis_name="s")
    out_t = [jax.ShapeDtypeStruct((BATCH * EMBED_DIM,), jnp.float32)] * 3
    return pl.kernel(
        _sc_body,
        mesh=mesh,
        out_type=out_t,
        scratch_types=[
            pltpu.VMEM((3 * BPW // 16, 16), jnp.int32),
            pltpu.VMEM((3 * NBINS * EMBED_DIM,), jnp.float32),
            pltpu.VMEM((2 * CHUNK * EMBED_DIM,), jnp.float32),
            pltpu.SemaphoreType.DMA,
            pltpu.SemaphoreType.DMA,
            pltpu.SemaphoreType.DMA,
        ],
    )(xr, helix_table.reshape(-1), sheet_table.reshape(-1),
      turns_table.reshape(-1))


BLK = 512


def _tc_body(x_ref, th, ts, tt, oh, os_, ot):
    idx = x_ref[...]
    iota6 = lax.broadcasted_iota(jnp.int32, (BLK, NBINS), 1)
    for col, (tab, out) in enumerate(((th, oh), (ts, os_), (tt, ot))):
        oneh = (idx[:, col].reshape(BLK, 1) == iota6).astype(jnp.float32)
        out[...] = jnp.dot(oneh, tab[...],
                           preferred_element_type=jnp.float32)


@jax.jit
def _tc_lookup(x, helix_table, sheet_table, turns_table):
    tab_spec = pl.BlockSpec((NBINS, EMBED_DIM), lambda i: (0, 0))
    row_spec = pl.BlockSpec((BLK, EMBED_DIM), lambda i: (i, 0))
    return pl.pallas_call(
        _tc_body,
        grid=(BATCH // BLK,),
        in_specs=[pl.BlockSpec((BLK, 3), lambda i: (i, 0)),
                  tab_spec, tab_spec, tab_spec],
        out_specs=[row_spec, row_spec, row_spec],
        out_shape=[jax.ShapeDtypeStruct((BATCH, EMBED_DIM), jnp.float32)] * 3,
    )(x, helix_table, sheet_table, turns_table)


def kernel(x, helix_table, sheet_table, turns_table):
    return tuple(_tc_lookup(x.astype(jnp.int32), helix_table, sheet_table,
                            turns_table))
